# baseline (speedup 1.0000x reference)
import functools

import jax
import jax.numpy as jnp
from jax import lax
from jax.experimental import pallas as pl
from jax.experimental.pallas import tpu as pltpu
from jax.experimental.pallas import tpu_sc as plsc

_N = 50000
_E = 800000
_F = 32
_R = 2

_LANES = 16
_NSUB = 16
_CH = 128
_GRP = 8
_EROWS = 6272
_ER_TILE = _EROWS // _NSUB
_GROUPS = _ER_TILE // _GRP
_SP_ROWS = 50176
_RPT = _SP_ROWS // _NSUB
_TRASH0 = _SP_ROWS
_SP_TOTAL = _SP_ROWS + _CH


def _load_group(edges, ebuf, row0, c_off, r):
    pltpu.sync_copy(edges.at[pl.ds(row0, _GRP)], ebuf)
    lanes = lax.iota(jnp.int32, _LANES)
    for j in range(_GRP):
        for q in range(_CH // _LANES):
            sl = pl.ds(q * _LANES, _LANES)
            trash = jnp.int32(_TRASH0 + j * _LANES) + lanes
            match = ebuf[j, 2, sl] == r
            ebuf[j, 0, sl] = jnp.where(match, ebuf[j, 0, sl] + c_off, c_off)
            ebuf[j, 1, sl] = jnp.where(match, ebuf[j, 1, sl], trash)


def _sc_body(tables, edges, zrows, out, sp, ebuf_a, ebuf_b,
             rows_0, rows_1, rows_2, rows_3, sem_0, sem_1, sem_2, sem_3):
    c = lax.axis_index("c")
    s = lax.axis_index("s")
    erow0 = s * _ER_TILE
    nrow0 = s * _RPT
    c_off = c * _N
    rows = (rows_0, rows_1, rows_2, rows_3)
    sems = (sem_0, sem_1, sem_2, sem_3)

    def fire(ebuf, j, k):
        pltpu.async_copy(tables.at[ebuf.at[j, 0]], rows[k % 4], sems[k % 4])

    def drain_scatter(ebuf, j, k):
        pltpu.make_async_copy(tables.at[ebuf.at[0, 0]], rows[k % 4],
                              sems[k % 4]).wait()
        pltpu.sync_copy(rows[k % 4], sp.at[ebuf.at[j, 1]], add=True)

    for r in (0, 1):
        combo = 2 * r + c
        pltpu.sync_copy(zrows, sp.at[pl.ds(nrow0, _RPT)])
        plsc.subcore_barrier()

        _load_group(edges, ebuf_a, erow0, c_off, r)
        fire(ebuf_a, 0, 0)
        fire(ebuf_a, 1, 1)
        fire(ebuf_a, 2, 2)

        def body(g2, carry):
            base = erow0 + g2 * 2 * _GRP
            _load_group(edges, ebuf_b, base + _GRP, c_off, r)
            for j in range(_GRP):
                if j < _GRP - 3:
                    fire(ebuf_a, j + 3, j + 3)
                else:
                    fire(ebuf_b, j + 3 - _GRP, j + 3)
                drain_scatter(ebuf_a, j, j)
            _load_group(edges, ebuf_a, base + 2 * _GRP, c_off, r)
            for j in range(_GRP):
                if j < _GRP - 3:
                    fire(ebuf_b, j + 3, j + 3)
                else:
                    fire(ebuf_a, j + 3 - _GRP, j + 3)
                drain_scatter(ebuf_b, j, j)
            return carry

        lax.fori_loop(0, (_GROUPS - 1) // 2, body, 0)
        for j in range(_GRP):
            if j < _GRP - 3:
                fire(ebuf_a, j + 3, j + 3)
            drain_scatter(ebuf_a, j, j)

        plsc.subcore_barrier()
        pltpu.sync_copy(sp.at[pl.ds(nrow0, _RPT)],
                        out.at[combo, pl.ds(nrow0, _RPT)])


_sc_accumulate = functools.partial(
    pl.kernel,
    mesh=plsc.VectorSubcoreMesh(core_axis_name="c", subcore_axis_name="s"),
    compiler_params=pltpu.CompilerParams(use_tc_tiling_on_sc=False),
    out_type=jax.ShapeDtypeStruct((2 * _R, _SP_ROWS, _F), jnp.float32),
    scratch_types=[
        pltpu.VMEM_SHARED((_SP_TOTAL, _F), jnp.float32),
        pltpu.VMEM((_GRP, 3, _CH), jnp.int32),
        pltpu.VMEM((_GRP, 3, _CH), jnp.int32),
        pltpu.VMEM((_CH, _F), jnp.float32),
        pltpu.VMEM((_CH, _F), jnp.float32),
        pltpu.VMEM((_CH, _F), jnp.float32),
        pltpu.VMEM((_CH, _F), jnp.float32),
        pltpu.SemaphoreType.DMA,
        pltpu.SemaphoreType.DMA,
        pltpu.SemaphoreType.DMA,
        pltpu.SemaphoreType.DMA,
    ],
)(_sc_body)


def _tc_fn(a0_ref, a1_ref, a2_ref, a3_ref, ww_ref, wt_ref, deg_ref,
           lab_ref, abl_ref):
    lab = jnp.dot(a1_ref[...], wt_ref[0], preferred_element_type=jnp.float32)
    lab = lab + jnp.dot(a3_ref[...], wt_ref[1], preferred_element_type=jnp.float32)
    lab_ref[...] = lab
    abl = jnp.dot(a0_ref[...], ww_ref[0], preferred_element_type=jnp.float32)
    abl = abl + jnp.dot(a2_ref[...], ww_ref[1], preferred_element_type=jnp.float32)
    abl_ref[...] = abl / deg_ref[...]


def kernel(labels, ability, deg, edge_index, edge_type, weight_worker, weight_task):
    tables = jnp.concatenate([labels, ability], axis=0)
    pad = _EROWS * _CH - _E
    srcr = jnp.pad(edge_index[0], (0, pad)).reshape(_EROWS, _CH)
    dstr = jnp.pad(edge_index[1], (0, pad)).reshape(_EROWS, _CH)
    typr = jnp.pad(edge_type, (0, pad), constant_values=2).reshape(_EROWS, _CH)
    edges = jnp.stack([srcr, dstr, typr], axis=1)
    zrows = jnp.zeros((_RPT, _F), jnp.float32)

    acc = _sc_accumulate(tables, edges, zrows)

    rows2d = _SP_ROWS * _F // 128
    acc2d = acc.reshape(4 * rows2d, 128)
    eye4 = jnp.eye(4, dtype=jnp.float32)
    ww4 = jnp.stack([jnp.kron(eye4, weight_worker[r]) for r in range(_R)])
    wt4 = jnp.stack([jnp.kron(eye4, weight_task[r]) for r in range(_R)])
    degp = jnp.pad(deg, ((0, _SP_ROWS - _N), (0, 0)))
    deg4 = jnp.repeat(degp.reshape(rows2d, 4), _F, axis=1)

    bl = 784
    grid = rows2d // bl
    specs = [pl.BlockSpec((bl, 128), lambda i, c=c: (c * grid + i, 0))
             for c in range(4)]
    lab2d, abl2d = pl.pallas_call(
        _tc_fn,
        grid=(grid,),
        in_specs=specs + [
            pl.BlockSpec((_R, 128, 128), lambda i: (0, 0, 0)),
            pl.BlockSpec((_R, 128, 128), lambda i: (0, 0, 0)),
            pl.BlockSpec((bl, 128), lambda i: (i, 0)),
        ],
        out_specs=[
            pl.BlockSpec((bl, 128), lambda i: (i, 0)),
            pl.BlockSpec((bl, 128), lambda i: (i, 0)),
        ],
        out_shape=[
            jax.ShapeDtypeStruct((rows2d, 128), jnp.float32),
            jax.ShapeDtypeStruct((rows2d, 128), jnp.float32),
        ],
    )(acc2d, acc2d, acc2d, acc2d, ww4, wt4, deg4)
    new_labels = lab2d.reshape(_SP_ROWS, _F)[:_N]
    new_ability = abl2d.reshape(_SP_ROWS, _F)[:_N]
    return (new_labels, new_ability)


# speedup vs baseline: 13.7070x; 13.7070x over previous
import functools

import jax
import jax.numpy as jnp
from jax import lax
from jax.experimental import pallas as pl
from jax.experimental.pallas import tpu as pltpu
from jax.experimental.pallas import tpu_sc as plsc

_N = 50000
_E = 800000
_F = 32
_R = 2

_LANES = 16
_NSUB = 16
_CH = 128
_GRP = 8
_EROWS = 6272
_ER_TILE = _EROWS // _NSUB
_GROUPS = _ER_TILE // _GRP
_SP_ROWS = 50176
_RPT = _SP_ROWS // _NSUB
_TRASH0 = _SP_ROWS
_SP_TOTAL = _SP_ROWS + _CH


def _load_group(edges, ebuf, row0, c_off, r):
    pltpu.sync_copy(edges.at[pl.ds(row0, _GRP)], ebuf)
    lanes = lax.iota(jnp.int32, _LANES)
    for j in range(_GRP):
        for q in range(_CH // _LANES):
            sl = pl.ds(q * _LANES, _LANES)
            trash = jnp.int32(_TRASH0 + j * _LANES) + lanes
            ebuf[j, 0, sl] = ebuf[j, 0, sl] + c_off
            ebuf[j, 1, sl] = jnp.where(ebuf[j, 2, sl] == r, ebuf[j, 1, sl], trash)


def _sc_body(tables, edges, zrows, out, sp, ebuf_a, ebuf_b,
             rows_0, rows_1, rows_2, rows_3, sem_0, sem_1, sem_2, sem_3):
    c = lax.axis_index("c")
    s = lax.axis_index("s")
    erow0 = s * _ER_TILE
    nrow0 = s * _RPT
    c_off = c * _N
    rows = (rows_0, rows_1, rows_2, rows_3)
    sems = (sem_0, sem_1, sem_2, sem_3)

    def fire(ebuf, j, k):
        pltpu.async_copy(tables.at[ebuf.at[j, 0]], rows[k % 4], sems[k % 4])

    def drain_scatter(ebuf, j, k):
        pltpu.make_async_copy(tables.at[ebuf.at[0, 0]], rows[k % 4],
                              sems[k % 4]).wait()
        pltpu.sync_copy(rows[k % 4], sp.at[ebuf.at[j, 1]], add=True)

    for r in (0, 1):
        combo = 2 * r + c
        pltpu.sync_copy(zrows, sp.at[pl.ds(nrow0, _RPT)])
        plsc.subcore_barrier()

        _load_group(edges, ebuf_a, erow0, c_off, r)
        fire(ebuf_a, 0, 0)
        fire(ebuf_a, 1, 1)
        fire(ebuf_a, 2, 2)

        def body(g2, carry):
            base = erow0 + g2 * 2 * _GRP
            _load_group(edges, ebuf_b, base + _GRP, c_off, r)
            for j in range(_GRP):
                if j < _GRP - 3:
                    fire(ebuf_a, j + 3, j + 3)
                else:
                    fire(ebuf_b, j + 3 - _GRP, j + 3)
                drain_scatter(ebuf_a, j, j)
            _load_group(edges, ebuf_a, base + 2 * _GRP, c_off, r)
            for j in range(_GRP):
                if j < _GRP - 3:
                    fire(ebuf_b, j + 3, j + 3)
                else:
                    fire(ebuf_a, j + 3 - _GRP, j + 3)
                drain_scatter(ebuf_b, j, j)
            return carry

        lax.fori_loop(0, (_GROUPS - 1) // 2, body, 0)
        for j in range(_GRP):
            if j < _GRP - 3:
                fire(ebuf_a, j + 3, j + 3)
            drain_scatter(ebuf_a, j, j)

        plsc.subcore_barrier()
        pltpu.sync_copy(sp.at[pl.ds(nrow0, _RPT)],
                        out.at[combo, pl.ds(nrow0, _RPT)])


_sc_accumulate = functools.partial(
    pl.kernel,
    mesh=plsc.VectorSubcoreMesh(core_axis_name="c", subcore_axis_name="s"),
    compiler_params=pltpu.CompilerParams(use_tc_tiling_on_sc=False),
    out_type=jax.ShapeDtypeStruct((2 * _R, _SP_ROWS, _F), jnp.float32),
    scratch_types=[
        pltpu.VMEM_SHARED((_SP_TOTAL, _F), jnp.float32),
        pltpu.VMEM((_GRP, 3, _CH), jnp.int32),
        pltpu.VMEM((_GRP, 3, _CH), jnp.int32),
        pltpu.VMEM((_CH, _F), jnp.float32),
        pltpu.VMEM((_CH, _F), jnp.float32),
        pltpu.VMEM((_CH, _F), jnp.float32),
        pltpu.VMEM((_CH, _F), jnp.float32),
        pltpu.SemaphoreType.DMA,
        pltpu.SemaphoreType.DMA,
        pltpu.SemaphoreType.DMA,
        pltpu.SemaphoreType.DMA,
    ],
)(_sc_body)


def _tc_fn(a0_ref, a1_ref, a2_ref, a3_ref, ww_ref, wt_ref, deg_ref,
           lab_ref, abl_ref):
    lab = jnp.dot(a1_ref[...], wt_ref[0], preferred_element_type=jnp.float32)
    lab = lab + jnp.dot(a3_ref[...], wt_ref[1], preferred_element_type=jnp.float32)
    lab_ref[...] = lab
    abl = jnp.dot(a0_ref[...], ww_ref[0], preferred_element_type=jnp.float32)
    abl = abl + jnp.dot(a2_ref[...], ww_ref[1], preferred_element_type=jnp.float32)
    abl_ref[...] = abl / deg_ref[...]


def kernel(labels, ability, deg, edge_index, edge_type, weight_worker, weight_task):
    tables = jnp.concatenate([labels, ability], axis=0)
    pad = _EROWS * _CH - _E
    srcr = jnp.pad(edge_index[0], (0, pad)).reshape(_EROWS, _CH)
    dstr = jnp.pad(edge_index[1], (0, pad)).reshape(_EROWS, _CH)
    typr = jnp.pad(edge_type, (0, pad), constant_values=2).reshape(_EROWS, _CH)
    edges = jnp.stack([srcr, dstr, typr], axis=1)
    zrows = jnp.zeros((_RPT, _F), jnp.float32)

    acc = _sc_accumulate(tables, edges, zrows)

    rows2d = _SP_ROWS * _F // 128
    acc2d = acc.reshape(4 * rows2d, 128)
    eye4 = jnp.eye(4, dtype=jnp.float32)
    ww4 = jnp.stack([jnp.kron(eye4, weight_worker[r]) for r in range(_R)])
    wt4 = jnp.stack([jnp.kron(eye4, weight_task[r]) for r in range(_R)])
    degp = jnp.pad(deg, ((0, _SP_ROWS - _N), (0, 0)))
    deg4 = jnp.repeat(degp.reshape(rows2d, 4), _F, axis=1)

    bl = 784
    grid = rows2d // bl
    specs = [pl.BlockSpec((bl, 128), lambda i, c=c: (c * grid + i, 0))
             for c in range(4)]
    lab2d, abl2d = pl.pallas_call(
        _tc_fn,
        grid=(grid,),
        in_specs=specs + [
            pl.BlockSpec((_R, 128, 128), lambda i: (0, 0, 0)),
            pl.BlockSpec((_R, 128, 128), lambda i: (0, 0, 0)),
            pl.BlockSpec((bl, 128), lambda i: (i, 0)),
        ],
        out_specs=[
            pl.BlockSpec((bl, 128), lambda i: (i, 0)),
            pl.BlockSpec((bl, 128), lambda i: (i, 0)),
        ],
        out_shape=[
            jax.ShapeDtypeStruct((rows2d, 128), jnp.float32),
            jax.ShapeDtypeStruct((rows2d, 128), jnp.float32),
        ],
    )(acc2d, acc2d, acc2d, acc2d, ww4, wt4, deg4)
    new_labels = lab2d.reshape(_SP_ROWS, _F)[:_N]
    new_ability = abl2d.reshape(_SP_ROWS, _F)[:_N]
    return (new_labels, new_ability)


# speedup vs baseline: 13.7557x; 1.0036x over previous
import functools

import jax
import jax.numpy as jnp
from jax import lax
from jax.experimental import pallas as pl
from jax.experimental.pallas import tpu as pltpu
from jax.experimental.pallas import tpu_sc as plsc

_N = 50000
_E = 800000
_F = 32
_R = 2

_LANES = 16
_NSUB = 16
_CH = 128
_GRP = 8
_EROWS = 6272
_ER_TILE = _EROWS // _NSUB
_GROUPS = _ER_TILE // _GRP
_SP_ROWS = 50176
_RPT = _SP_ROWS // _NSUB
_TRASH0 = _SP_ROWS
_SP_TOTAL = _SP_ROWS + _CH


def _load_group(edges, ebuf, row0, c_off, r):
    pltpu.sync_copy(edges.at[pl.ds(row0, _GRP)], ebuf)
    lanes = lax.iota(jnp.int32, _LANES)
    for j in range(_GRP):
        for q in range(_CH // _LANES):
            sl = pl.ds(q * _LANES, _LANES)
            trash = jnp.int32(_TRASH0 + j * _LANES) + lanes
            ebuf[j, 0, sl] = ebuf[j, 0, sl] + c_off
            ebuf[j, 1, sl] = jnp.where(ebuf[j, 2, sl] == r, ebuf[j, 1, sl], trash)


def _sc_body(tables, edges, zrows, out, sp, ebuf_a, ebuf_b,
             rows_0, rows_1, rows_2, rows_3, sem_0, sem_1, sem_2, sem_3):
    c = lax.axis_index("c")
    s = lax.axis_index("s")
    erow0 = s * _ER_TILE
    nrow0 = s * _RPT
    c_off = c * _N
    rows = (rows_0, rows_1, rows_2, rows_3)
    sems = (sem_0, sem_1, sem_2, sem_3)

    def fire(ebuf, j, k):
        pltpu.async_copy(tables.at[ebuf.at[j, 0]], rows[k % 4], sems[k % 4])

    def drain_scatter(ebuf, j, k):
        pltpu.make_async_copy(tables.at[ebuf.at[0, 0]], rows[k % 4],
                              sems[k % 4]).wait()
        pltpu.sync_copy(rows[k % 4], sp.at[ebuf.at[j, 1]], add=True)

    for r in (0, 1):
        combo = 2 * r + c
        pltpu.sync_copy(zrows, sp.at[pl.ds(nrow0, _RPT)])
        plsc.subcore_barrier()

        _load_group(edges, ebuf_a, erow0, c_off, r)
        fire(ebuf_a, 0, 0)
        fire(ebuf_a, 1, 1)
        fire(ebuf_a, 2, 2)
        fire(ebuf_a, 3, 3)

        def body(g2, carry):
            base = erow0 + g2 * 2 * _GRP
            _load_group(edges, ebuf_b, base + _GRP, c_off, r)
            for j in range(_GRP):
                drain_scatter(ebuf_a, j, j)
                if j < _GRP - 4:
                    fire(ebuf_a, j + 4, j)
                else:
                    fire(ebuf_b, j + 4 - _GRP, j)
            _load_group(edges, ebuf_a, base + 2 * _GRP, c_off, r)
            for j in range(_GRP):
                drain_scatter(ebuf_b, j, j)
                if j < _GRP - 4:
                    fire(ebuf_b, j + 4, j)
                else:
                    fire(ebuf_a, j + 4 - _GRP, j)
            return carry

        lax.fori_loop(0, (_GROUPS - 1) // 2, body, 0)
        for j in range(_GRP):
            drain_scatter(ebuf_a, j, j)
            if j < _GRP - 4:
                fire(ebuf_a, j + 4, j)

        plsc.subcore_barrier()
        pltpu.sync_copy(sp.at[pl.ds(nrow0, _RPT)],
                        out.at[combo, pl.ds(nrow0, _RPT)])


_sc_accumulate = functools.partial(
    pl.kernel,
    mesh=plsc.VectorSubcoreMesh(core_axis_name="c", subcore_axis_name="s"),
    compiler_params=pltpu.CompilerParams(use_tc_tiling_on_sc=False),
    out_type=jax.ShapeDtypeStruct((2 * _R, _SP_ROWS, _F), jnp.float32),
    scratch_types=[
        pltpu.VMEM_SHARED((_SP_TOTAL, _F), jnp.float32),
        pltpu.VMEM((_GRP, 3, _CH), jnp.int32),
        pltpu.VMEM((_GRP, 3, _CH), jnp.int32),
        pltpu.VMEM((_CH, _F), jnp.float32),
        pltpu.VMEM((_CH, _F), jnp.float32),
        pltpu.VMEM((_CH, _F), jnp.float32),
        pltpu.VMEM((_CH, _F), jnp.float32),
        pltpu.SemaphoreType.DMA,
        pltpu.SemaphoreType.DMA,
        pltpu.SemaphoreType.DMA,
        pltpu.SemaphoreType.DMA,
    ],
)(_sc_body)


def _tc_fn(a0_ref, a1_ref, a2_ref, a3_ref, ww_ref, wt_ref, deg_ref,
           lab_ref, abl_ref):
    lab = jnp.dot(a1_ref[...], wt_ref[0], preferred_element_type=jnp.float32)
    lab = lab + jnp.dot(a3_ref[...], wt_ref[1], preferred_element_type=jnp.float32)
    lab_ref[...] = lab
    abl = jnp.dot(a0_ref[...], ww_ref[0], preferred_element_type=jnp.float32)
    abl = abl + jnp.dot(a2_ref[...], ww_ref[1], preferred_element_type=jnp.float32)
    abl_ref[...] = abl / deg_ref[...]


def kernel(labels, ability, deg, edge_index, edge_type, weight_worker, weight_task):
    tables = jnp.concatenate([labels, ability], axis=0)
    pad = _EROWS * _CH - _E
    srcr = jnp.pad(edge_index[0], (0, pad)).reshape(_EROWS, _CH)
    dstr = jnp.pad(edge_index[1], (0, pad)).reshape(_EROWS, _CH)
    typr = jnp.pad(edge_type, (0, pad), constant_values=2).reshape(_EROWS, _CH)
    edges = jnp.stack([srcr, dstr, typr], axis=1)
    zrows = jnp.zeros((_RPT, _F), jnp.float32)

    acc = _sc_accumulate(tables, edges, zrows)

    rows2d = _SP_ROWS * _F // 128
    acc2d = acc.reshape(4 * rows2d, 128)
    eye4 = jnp.eye(4, dtype=jnp.float32)
    ww4 = jnp.stack([jnp.kron(eye4, weight_worker[r]) for r in range(_R)])
    wt4 = jnp.stack([jnp.kron(eye4, weight_task[r]) for r in range(_R)])
    degp = jnp.pad(deg, ((0, _SP_ROWS - _N), (0, 0)))
    deg4 = jnp.repeat(degp.reshape(rows2d, 4), _F, axis=1)

    bl = 784
    grid = rows2d // bl
    specs = [pl.BlockSpec((bl, 128), lambda i, c=c: (c * grid + i, 0))
             for c in range(4)]
    lab2d, abl2d = pl.pallas_call(
        _tc_fn,
        grid=(grid,),
        in_specs=specs + [
            pl.BlockSpec((_R, 128, 128), lambda i: (0, 0, 0)),
            pl.BlockSpec((_R, 128, 128), lambda i: (0, 0, 0)),
            pl.BlockSpec((bl, 128), lambda i: (i, 0)),
        ],
        out_specs=[
            pl.BlockSpec((bl, 128), lambda i: (i, 0)),
            pl.BlockSpec((bl, 128), lambda i: (i, 0)),
        ],
        out_shape=[
            jax.ShapeDtypeStruct((rows2d, 128), jnp.float32),
            jax.ShapeDtypeStruct((rows2d, 128), jnp.float32),
        ],
    )(acc2d, acc2d, acc2d, acc2d, ww4, wt4, deg4)
    new_labels = lab2d.reshape(_SP_ROWS, _F)[:_N]
    new_ability = abl2d.reshape(_SP_ROWS, _F)[:_N]
    return (new_labels, new_ability)


# speedup vs baseline: 15.1943x; 1.1046x over previous
import functools

import jax
import jax.numpy as jnp
from jax import lax
from jax.experimental import pallas as pl
from jax.experimental.pallas import tpu as pltpu
from jax.experimental.pallas import tpu_sc as plsc

_N = 50000
_E = 800000
_F = 32
_R = 2

_LANES = 16
_NSUB = 16
_CH = 128
_GRP = 8
_EROWS = 6272
_ER_TILE = _EROWS // _NSUB
_GROUPS = _ER_TILE // _GRP
_SP_ROWS = 50176
_RPT = _SP_ROWS // _NSUB
_TRASH0 = _SP_ROWS
_SP_TOTAL = _SP_ROWS + _CH


def _start_load(edges, ebuf, lsem, row0):
    pltpu.async_copy(edges.at[pl.ds(row0, _GRP)], ebuf, lsem)


def _finish_load(edges, ebuf, lsem, c_off, r):
    pltpu.make_async_copy(edges.at[pl.ds(0, _GRP)], ebuf, lsem).wait()
    lanes = lax.iota(jnp.int32, _LANES)
    for j in range(_GRP):
        for q in range(_CH // _LANES):
            sl = pl.ds(q * _LANES, _LANES)
            trash = jnp.int32(_TRASH0 + j * _LANES) + lanes
            ebuf[j, 0, sl] = ebuf[j, 0, sl] + c_off
            ebuf[j, 1, sl] = jnp.where(ebuf[j, 2, sl] == r, ebuf[j, 1, sl], trash)


def _sc_body(tables, edges, zrows, out, sp, ebuf_a, ebuf_b, ebuf_c,
             rows_0, rows_1, rows_2, rows_3, lsem_a, lsem_b, lsem_c,
             sem_0, sem_1, sem_2, sem_3):
    c = lax.axis_index("c")
    s = lax.axis_index("s")
    erow0 = s * _ER_TILE
    nrow0 = s * _RPT
    c_off = c * _N
    ebufs = (ebuf_a, ebuf_b, ebuf_c)
    lsems = (lsem_a, lsem_b, lsem_c)
    rows = (rows_0, rows_1, rows_2, rows_3)
    sems = (sem_0, sem_1, sem_2, sem_3)

    def fire(ebuf, j, k):
        pltpu.async_copy(tables.at[ebuf.at[j, 0]], rows[k % 4], sems[k % 4])

    def drain_scatter(ebuf, j, k):
        pltpu.make_async_copy(tables.at[ebuf.at[0, 0]], rows[k % 4],
                              sems[k % 4]).wait()
        pltpu.sync_copy(rows[k % 4], sp.at[ebuf.at[j, 1]], add=True)

    for r in (0, 1):
        combo = 2 * r + c
        pltpu.sync_copy(zrows, sp.at[pl.ds(nrow0, _RPT)])
        plsc.subcore_barrier()

        _start_load(edges, ebuf_a, lsem_a, erow0)
        _start_load(edges, ebuf_b, lsem_b, erow0 + _GRP)
        _finish_load(edges, ebuf_a, lsem_a, c_off, r)
        fire(ebuf_a, 0, 0)
        fire(ebuf_a, 1, 1)
        fire(ebuf_a, 2, 2)
        fire(ebuf_a, 3, 3)

        def body(i, carry):
            for t in range(3):
                cur, nxt, pre = t % 3, (t + 1) % 3, (t + 2) % 3
                base = erow0 + (3 * i + t) * _GRP
                _finish_load(edges, ebufs[nxt], lsems[nxt], c_off, r)
                if t < 2:
                    _start_load(edges, ebufs[pre], lsems[pre], base + 2 * _GRP)
                else:
                    @pl.when(i < (_GROUPS - 1) // 3 - 1)
                    def _():
                        _start_load(edges, ebufs[pre], lsems[pre],
                                    base + 2 * _GRP)
                for j in range(_GRP):
                    drain_scatter(ebufs[cur], j, j)
                    if j < _GRP - 4:
                        fire(ebufs[cur], j + 4, j)
                    else:
                        fire(ebufs[nxt], j + 4 - _GRP, j)
            return carry

        lax.fori_loop(0, (_GROUPS - 1) // 3, body, 0)
        for j in range(_GRP):
            drain_scatter(ebuf_a, j, j)
            if j < _GRP - 4:
                fire(ebuf_a, j + 4, j)

        plsc.subcore_barrier()
        pltpu.sync_copy(sp.at[pl.ds(nrow0, _RPT)],
                        out.at[combo, pl.ds(nrow0, _RPT)])


_sc_accumulate = functools.partial(
    pl.kernel,
    mesh=plsc.VectorSubcoreMesh(core_axis_name="c", subcore_axis_name="s"),
    compiler_params=pltpu.CompilerParams(use_tc_tiling_on_sc=False),
    out_type=jax.ShapeDtypeStruct((2 * _R, _SP_ROWS, _F), jnp.float32),
    scratch_types=[
        pltpu.VMEM_SHARED((_SP_TOTAL, _F), jnp.float32),
        pltpu.VMEM((_GRP, 3, _CH), jnp.int32),
        pltpu.VMEM((_GRP, 3, _CH), jnp.int32),
        pltpu.VMEM((_GRP, 3, _CH), jnp.int32),
        pltpu.VMEM((_CH, _F), jnp.float32),
        pltpu.VMEM((_CH, _F), jnp.float32),
        pltpu.VMEM((_CH, _F), jnp.float32),
        pltpu.VMEM((_CH, _F), jnp.float32),
        pltpu.SemaphoreType.DMA,
        pltpu.SemaphoreType.DMA,
        pltpu.SemaphoreType.DMA,
        pltpu.SemaphoreType.DMA,
        pltpu.SemaphoreType.DMA,
        pltpu.SemaphoreType.DMA,
        pltpu.SemaphoreType.DMA,
    ],
)(_sc_body)


def _tc_fn(a0_ref, a1_ref, a2_ref, a3_ref, ww_ref, wt_ref, deg_ref,
           lab_ref, abl_ref):
    lab = jnp.dot(a1_ref[...], wt_ref[0], preferred_element_type=jnp.float32)
    lab = lab + jnp.dot(a3_ref[...], wt_ref[1], preferred_element_type=jnp.float32)
    lab_ref[...] = lab
    abl = jnp.dot(a0_ref[...], ww_ref[0], preferred_element_type=jnp.float32)
    abl = abl + jnp.dot(a2_ref[...], ww_ref[1], preferred_element_type=jnp.float32)
    abl_ref[...] = abl / deg_ref[...]


def kernel(labels, ability, deg, edge_index, edge_type, weight_worker, weight_task):
    tables = jnp.concatenate([labels, ability], axis=0)
    pad = _EROWS * _CH - _E
    srcr = jnp.pad(edge_index[0], (0, pad)).reshape(_EROWS, _CH)
    dstr = jnp.pad(edge_index[1], (0, pad)).reshape(_EROWS, _CH)
    typr = jnp.pad(edge_type, (0, pad), constant_values=2).reshape(_EROWS, _CH)
    edges = jnp.stack([srcr, dstr, typr], axis=1)
    zrows = jnp.zeros((_RPT, _F), jnp.float32)

    acc = _sc_accumulate(tables, edges, zrows)

    rows2d = _SP_ROWS * _F // 128
    acc2d = acc.reshape(4 * rows2d, 128)
    eye4 = jnp.eye(4, dtype=jnp.float32)
    ww4 = jnp.stack([jnp.kron(eye4, weight_worker[r]) for r in range(_R)])
    wt4 = jnp.stack([jnp.kron(eye4, weight_task[r]) for r in range(_R)])
    degp = jnp.pad(deg, ((0, _SP_ROWS - _N), (0, 0)))
    deg4 = jnp.repeat(degp.reshape(rows2d, 4), _F, axis=1)

    bl = 784
    grid = rows2d // bl
    specs = [pl.BlockSpec((bl, 128), lambda i, c=c: (c * grid + i, 0))
             for c in range(4)]
    lab2d, abl2d = pl.pallas_call(
        _tc_fn,
        grid=(grid,),
        in_specs=specs + [
            pl.BlockSpec((_R, 128, 128), lambda i: (0, 0, 0)),
            pl.BlockSpec((_R, 128, 128), lambda i: (0, 0, 0)),
            pl.BlockSpec((bl, 128), lambda i: (i, 0)),
        ],
        out_specs=[
            pl.BlockSpec((bl, 128), lambda i: (i, 0)),
            pl.BlockSpec((bl, 128), lambda i: (i, 0)),
        ],
        out_shape=[
            jax.ShapeDtypeStruct((rows2d, 128), jnp.float32),
            jax.ShapeDtypeStruct((rows2d, 128), jnp.float32),
        ],
    )(acc2d, acc2d, acc2d, acc2d, ww4, wt4, deg4)
    new_labels = lab2d.reshape(_SP_ROWS, _F)[:_N]
    new_ability = abl2d.reshape(_SP_ROWS, _F)[:_N]
    return (new_labels, new_ability)
